# baseline (speedup 1.0000x reference)
import functools
import jax
import jax.numpy as jnp
from jax import lax
from jax.experimental import pallas as pl
from jax.experimental.pallas import tpu as pltpu
from jax.experimental.pallas import tpu_sc as plsc

N, E, F, H, G = 10000, 320000, 128, 128, 64
HH = H // 2
NC, NS = 2, 16
NW = NC * NS
CH = 128
NT = 80
EPAD = NW * NT * CH
NP = NW * 320
RPT = 624
ZR = 208
RB = 1000


def _mesh():
    return plsc.VectorSubcoreMesh(core_axis_name="c", subcore_axis_name="s",
                                  num_cores=NC, num_subcores=NS)


def _deg_body(dsti, ew, out, dstall, ewall, zbuf, acc, ssem):
    cid = lax.axis_index("c")
    sid = lax.axis_index("s")
    wid = cid * NS + sid

    pltpu.sync_copy(dsti.at[wid], dstall)
    pltpu.sync_copy(ew.at[wid], ewall)

    @plsc.parallel_loop(0, 64, 1, unroll=2)
    def _zrow(r):
        zbuf[pl.ds(16 * r, 16)] = jnp.zeros((16,), jnp.float32)

    @pl.when(sid < 10)
    def _():
        pltpu.sync_copy(zbuf.at[pl.ds(0, 1000)], acc.at[pl.ds(sid * 1000, 1000)])
    plsc.subcore_barrier()

    def scat(ch, _):
        pltpu.async_copy(ewall.at[ch], acc.at[dstall.at[ch]], ssem, add=True)
        return 0
    lax.fori_loop(0, NT, scat, 0)

    def drain(ch, _):
        pltpu.make_async_copy(ewall.at[ch], acc.at[dstall.at[ch]], ssem).wait()
        return 0
    lax.fori_loop(0, NT, drain, 0)
    plsc.subcore_barrier()

    @pl.when(sid < 10)
    def _():
        pltpu.sync_copy(acc.at[pl.ds(sid * 1000, 1000)], zbuf.at[pl.ds(0, 1000)])
        pltpu.sync_copy(zbuf.at[pl.ds(0, 1000)],
                        out.at[pl.ds(cid * N + sid * 1000, 1000)])


def _deg_call(dsti, ew):
    return pl.kernel(
        _deg_body,
        out_type=jax.ShapeDtypeStruct((NC * N,), jnp.float32),
        mesh=_mesh(),
        compiler_params=pltpu.CompilerParams(use_tc_tiling_on_sc=False),
        scratch_types=[
            pltpu.VMEM((NT, CH), jnp.int32),
            pltpu.VMEM((NT, CH), jnp.float32),
            pltpu.VMEM((1024,), jnp.float32),
            pltpu.VMEM_SHARED((N,), jnp.float32),
            pltpu.SemaphoreType.DMA,
        ],
    )(dsti, ew)


NB = 4


def _edge_body(plo, phi, srci, dsti, ew, out0, out1, srcall, dstall, ewall,
               rows, zbuf, stg, acc, g0, g1, g2, g3, s0, s1, s2, s3):
    cid = lax.axis_index("c")
    sid = lax.axis_index("s")
    wid = cid * NS + sid
    phalves = (plo, phi)
    outs = (out0, out1)
    gsems = (g0, g1, g2, g3)
    ssems = (s0, s1, s2, s3)

    pltpu.sync_copy(srci.at[wid], srcall)
    pltpu.sync_copy(dsti.at[wid], dstall)
    pltpu.sync_copy(ew.at[wid], ewall)

    @plsc.parallel_loop(0, ZR, 1, unroll=4)
    def _zrow(r):
        for k in range(HH // 16):
            zbuf[r, pl.ds(16 * k, 16)] = jnp.zeros((16,), jnp.float32)

    def scale(b, ch):
        @plsc.parallel_loop(0, CH // 16, 1, unroll=4)
        def _sgroup(g):
            wvec = ewall[ch, pl.ds(16 * g, 16)]
            for i in range(16):
                w = wvec[i]
                r = 16 * g + i
                for k in range(HH // 16):
                    sl = pl.ds(16 * k, 16)
                    rows[b, r, sl] = rows[b, r, sl] * w

    for ph in range(2):
        p_hbm = phalves[ph]
        out = outs[ph]

        for j in range(3):
            pltpu.sync_copy(zbuf, acc.at[pl.ds(sid * RPT + j * ZR, ZR)])

        @pl.when(sid == NS - 1)
        def _():
            pltpu.sync_copy(zbuf.at[pl.ds(0, 16)], acc.at[pl.ds(NS * RPT, 16)])
        plsc.subcore_barrier()

        def start_gather(b, ch):
            pltpu.async_copy(p_hbm.at[srcall.at[ch]], rows.at[b], gsems[b])

        def wait_gather(b, ch):
            pltpu.make_async_copy(p_hbm.at[srcall.at[ch]], rows.at[b],
                                  gsems[b]).wait()

        def start_scatter(b, ch):
            pltpu.async_copy(rows.at[b], acc.at[dstall.at[ch]], ssems[b],
                             add=True)

        def wait_scatter(b, ch):
            pltpu.make_async_copy(rows.at[b], acc.at[dstall.at[ch]],
                                  ssems[b]).wait()

        for b in range(NB - 1):
            start_gather(b, b)

        def step(j, _):
            for b in range(NB):
                ch = j * NB + b
                chf = ch + NB - 1
                slotf = (b + NB - 1) % NB

                @pl.when(jnp.logical_and(chf >= NB, chf < NT))
                def _():
                    wait_scatter(slotf, chf - NB)

                @pl.when(chf < NT)
                def _():
                    start_gather(slotf, chf)
                wait_gather(b, ch)
                scale(b, ch)
                start_scatter(b, ch)
            return 0
        lax.fori_loop(0, NT // NB, step, 0)
        for b in range(NB):
            wait_scatter(b, NT - NB + b)
        plsc.subcore_barrier()
        for j in range(3):
            pltpu.sync_copy(acc.at[pl.ds(sid * RPT + j * ZR, ZR)], stg)
            pltpu.sync_copy(stg, out.at[cid, pl.ds(sid * RPT + j * ZR, ZR)])

        @pl.when(sid == NS - 1)
        def _():
            pltpu.sync_copy(acc.at[pl.ds(NS * RPT, 16)], stg.at[pl.ds(0, 16)])
            pltpu.sync_copy(stg.at[pl.ds(0, 16)],
                            out.at[cid, pl.ds(NS * RPT, 16)])


def _edge_call(plo, phi, srci, dsti, ew):
    return pl.kernel(
        _edge_body,
        out_type=[
            jax.ShapeDtypeStruct((NC, N, HH), jnp.float32),
            jax.ShapeDtypeStruct((NC, N, HH), jnp.float32),
        ],
        mesh=_mesh(),
        compiler_params=pltpu.CompilerParams(use_tc_tiling_on_sc=False),
        scratch_types=[
            pltpu.VMEM((NT, CH), jnp.int32),
            pltpu.VMEM((NT, CH), jnp.int32),
            pltpu.VMEM((NT, CH), jnp.float32),
            pltpu.VMEM((NB, CH, HH), jnp.float32),
            pltpu.VMEM((ZR, HH), jnp.float32),
            pltpu.VMEM((ZR, HH), jnp.float32),
            pltpu.VMEM_SHARED((N, HH), jnp.float32),
        ] + [pltpu.SemaphoreType.DMA] * (2 * NB),
    )(plo, phi, srci, dsti, ew)


def _pool_body(h0, h1, h2, batch, maxo, sumo, batchv, hbuf, maxacc, sumacc):
    cid = lax.axis_index("c")
    sid = lax.axis_index("s")
    wid = cid * NS + sid
    base = wid * 320
    hs = (h0, h1, h2)

    neg = jnp.full((16,), -jnp.inf, dtype=jnp.float32)
    zero = jnp.zeros((16,), jnp.float32)

    def irow(r, _):
        for k in range(8):
            sl = pl.ds(16 * k, 16)
            maxacc[r, sl] = neg
            sumacc[r, sl] = zero
        return 0
    lax.fori_loop(0, 3 * G, irow, 0)

    for sub in range(4):
        @pl.when(base + sub * 80 < N)
        def _():
            _pool_sub(batch, hs, batchv, hbuf, maxacc, sumacc, base, sub)

    pltpu.sync_copy(maxacc, maxo.at[wid])
    pltpu.sync_copy(sumacc, sumo.at[wid])


def _pool_sub(batch, hs, batchv, hbuf, maxacc, sumacc, base, sub):
        pltpu.sync_copy(batch.at[pl.ds(base + sub * 80, 80)],
                        batchv.at[pl.ds(sub * 80, 80)])
        for l in range(3):
            pltpu.sync_copy(hs[l].at[pl.ds(base + sub * 80, 80)], hbuf.at[l])

        def rowgrp(g, _):
            bvec = batchv[pl.ds(sub * 80 + 16 * g, 16)]
            for i in range(16):
                b = bvec[i]
                r = 16 * g + i
                for l in range(3):
                    idx = b + l * G
                    for k in range(8):
                        sl = pl.ds(16 * k, 16)
                        hv = hbuf[l, r, sl]
                        maxacc[idx, sl] = jnp.maximum(maxacc[idx, sl], hv)
                        sumacc[idx, sl] = sumacc[idx, sl] + hv
            return 0
        lax.fori_loop(0, 5, rowgrp, 0)


def _pool_call(h0, h1, h2, batch):
    return pl.kernel(
        _pool_body,
        out_type=[
            jax.ShapeDtypeStruct((NW, 3 * G, H), jnp.float32),
            jax.ShapeDtypeStruct((NW, 3 * G, H), jnp.float32),
        ],
        mesh=_mesh(),
        scratch_types=[
            pltpu.VMEM((320,), jnp.int32),
            pltpu.VMEM((3, 80, H), jnp.float32),
            pltpu.VMEM((3 * G, H), jnp.float32),
            pltpu.VMEM((3 * G, H), jnp.float32),
        ],
    )(h0, h1, h2, batch)


def _mlph_body(x_ref, w1_ref, b1_ref, w2_ref, b2_ref, h_ref):
    h = jnp.maximum(x_ref[...] @ w1_ref[...] + b1_ref[...], 0.0)
    h_ref[...] = jnp.maximum(h @ w2_ref[...] + b2_ref[...], 0.0)


def _mlph_call(x, W1, b1, W2, b2):
    return pl.pallas_call(
        _mlph_body,
        grid=(N // RB,),
        in_specs=[
            pl.BlockSpec((RB, F), lambda i: (i, 0)),
            pl.BlockSpec((F, H), lambda i: (0, 0)),
            pl.BlockSpec((1, H), lambda i: (0, 0)),
            pl.BlockSpec((H, H), lambda i: (0, 0)),
            pl.BlockSpec((1, H), lambda i: (0, 0)),
        ],
        out_specs=pl.BlockSpec((RB, H), lambda i: (i, 0)),
        out_shape=jax.ShapeDtypeStruct((N, H), jnp.float32),
    )(x, W1, b1.reshape(1, H), W2, b2.reshape(1, H))


def _proj_body(h_ref, wc_ref, deg_ref, plo_ref, phi_ref, dinv_ref):
    deg = deg_ref[0] + deg_ref[1] + 1.0
    dinv = lax.rsqrt(deg)
    p = dinv * (h_ref[...] @ wc_ref[...])
    plo_ref[...] = p[:, :HH]
    phi_ref[...] = p[:, HH:]
    dinv_ref[...] = dinv


def _proj_call(h, Wc0, deg2):
    return pl.pallas_call(
        _proj_body,
        grid=(N // RB,),
        in_specs=[
            pl.BlockSpec((RB, H), lambda i: (i, 0)),
            pl.BlockSpec((H, H), lambda i: (0, 0)),
            pl.BlockSpec((2, RB, 1), lambda i: (0, i, 0)),
        ],
        out_specs=[
            pl.BlockSpec((RB, HH), lambda i: (i, 0)),
            pl.BlockSpec((RB, HH), lambda i: (i, 0)),
            pl.BlockSpec((RB, 1), lambda i: (i, 0)),
        ],
        out_shape=[
            jax.ShapeDtypeStruct((N, HH), jnp.float32),
            jax.ShapeDtypeStruct((N, HH), jnp.float32),
            jax.ShapeDtypeStruct((N, 1), jnp.float32),
        ],
    )(h, Wc0, deg2)


def _combine_body(plo_part, phi_part, plo_ref, phi_ref, dinv_ref, bc_ref,
                  wcn_ref, h_ref, plon_ref, phin_ref):
    dinv = dinv_ref[...]
    out_lo = dinv * (plo_part[0] + plo_part[1] + plo_ref[...]) + bc_ref[:, :HH]
    out_hi = dinv * (phi_part[0] + phi_part[1] + phi_ref[...]) + bc_ref[:, HH:]
    h = jnp.concatenate([jnp.maximum(out_lo, 0.0), jnp.maximum(out_hi, 0.0)],
                        axis=1)
    h_ref[...] = h
    pn = dinv * (h @ wcn_ref[...])
    plon_ref[...] = pn[:, :HH]
    phin_ref[...] = pn[:, HH:]


def _combine_call(part_lo, part_hi, plo, phi, dinv, bc, wcn):
    return pl.pallas_call(
        _combine_body,
        grid=(N // RB,),
        in_specs=[
            pl.BlockSpec((2, RB, HH), lambda i: (0, i, 0)),
            pl.BlockSpec((2, RB, HH), lambda i: (0, i, 0)),
            pl.BlockSpec((RB, HH), lambda i: (i, 0)),
            pl.BlockSpec((RB, HH), lambda i: (i, 0)),
            pl.BlockSpec((RB, 1), lambda i: (i, 0)),
            pl.BlockSpec((1, H), lambda i: (0, 0)),
            pl.BlockSpec((H, H), lambda i: (0, 0)),
        ],
        out_specs=[
            pl.BlockSpec((RB, H), lambda i: (i, 0)),
            pl.BlockSpec((RB, HH), lambda i: (i, 0)),
            pl.BlockSpec((RB, HH), lambda i: (i, 0)),
        ],
        out_shape=[
            jax.ShapeDtypeStruct((N, H), jnp.float32),
            jax.ShapeDtypeStruct((N, HH), jnp.float32),
            jax.ShapeDtypeStruct((N, HH), jnp.float32),
        ],
    )(part_lo, part_hi, plo, phi, dinv, bc.reshape(1, H), wcn)


def _poolcomb_body(maxp_ref, sump_ref, batch_ref, out_ref):
    mx = jnp.max(maxp_ref[...], axis=0)
    mx = jnp.where(jnp.isfinite(mx), mx, 0.0)
    sums = jnp.sum(sump_ref[...], axis=0)
    ids = lax.broadcasted_iota(jnp.int32, (1, G), 1)
    cnt = jnp.sum((batch_ref[...] == ids).astype(jnp.float32), axis=0)
    mean = sums / jnp.maximum(cnt, 1.0)[:, None]
    out_ref[...] = jnp.concatenate([mx, mean], axis=1)


def _poolcomb_call(maxp, sump, batch2):
    return pl.pallas_call(
        _poolcomb_body,
        grid=(3,),
        in_specs=[
            pl.BlockSpec((NW, G, H), lambda l: (0, l, 0)),
            pl.BlockSpec((NW, G, H), lambda l: (0, l, 0)),
            pl.BlockSpec((N, 1), lambda l: (0, 0)),
        ],
        out_specs=pl.BlockSpec((G, 2 * H), lambda l: (0, l)),
        out_shape=jax.ShapeDtypeStruct((G, 6 * H), jnp.float32),
    )(maxp, sump, batch2)


def kernel(x, edge_index, edge_attr, batch, W1, b1, W2, b2, Wc0, bc0,
           Wc1, bc1, Wc2, bc2):
    pad = EPAD - E
    pidx = jnp.arange(pad, dtype=jnp.int32) % N
    srci = jnp.concatenate([edge_index[0], pidx])
    dsti = jnp.concatenate([edge_index[1], pidx])
    ewp = jnp.concatenate([edge_attr, jnp.zeros((pad,), jnp.float32)])
    srci3 = srci.reshape(NW, NT, CH)
    dsti3 = dsti.reshape(NW, NT, CH)
    ewp3 = ewp.reshape(NW, NT, CH)

    h0 = _mlph_call(x, W1, b1, W2, b2)
    deg2 = _deg_call(dsti3, ewp3).reshape(NC, N, 1)
    plo, phi, dinv = _proj_call(h0, Wc0, deg2)

    hs = []
    for bc, wcn in ((bc0, Wc1), (bc1, Wc2), (bc2, Wc2)):
        part_lo, part_hi = _edge_call(plo, phi, srci3, dsti3, ewp3)
        h, plo, phi = _combine_call(part_lo, part_hi, plo, phi, dinv, bc, wcn)
        hs.append(h)

    maxp, sump = _pool_call(hs[0], hs[1], hs[2], batch)
    return _poolcomb_call(maxp, sump, batch.reshape(N, 1))


# speedup vs baseline: 1.0040x; 1.0040x over previous
import functools
import jax
import jax.numpy as jnp
from jax import lax
from jax.experimental import pallas as pl
from jax.experimental.pallas import tpu as pltpu
from jax.experimental.pallas import tpu_sc as plsc

N, E, F, H, G = 10000, 320000, 128, 128, 64
HH = H // 2
NC, NS = 2, 16
NW = NC * NS
CH = 128
NT = 80
EPAD = NW * NT * CH
NP = NW * 320
RPT = 624
ZR = 208
RB = 1000


def _mesh():
    return plsc.VectorSubcoreMesh(core_axis_name="c", subcore_axis_name="s",
                                  num_cores=NC, num_subcores=NS)


def _deg_body(dsti, ew, out, dstall, ewall, zbuf, acc, ssem):
    cid = lax.axis_index("c")
    sid = lax.axis_index("s")
    wid = cid * NS + sid

    pltpu.sync_copy(dsti.at[wid], dstall)
    pltpu.sync_copy(ew.at[wid], ewall)

    @plsc.parallel_loop(0, 64, 1, unroll=2)
    def _zrow(r):
        zbuf[pl.ds(16 * r, 16)] = jnp.zeros((16,), jnp.float32)

    @pl.when(sid < 10)
    def _():
        pltpu.sync_copy(zbuf.at[pl.ds(0, 1000)], acc.at[pl.ds(sid * 1000, 1000)])
    plsc.subcore_barrier()

    def scat(ch, _):
        pltpu.async_copy(ewall.at[ch], acc.at[dstall.at[ch]], ssem, add=True)
        return 0
    lax.fori_loop(0, NT, scat, 0)

    def drain(ch, _):
        pltpu.make_async_copy(ewall.at[ch], acc.at[dstall.at[ch]], ssem).wait()
        return 0
    lax.fori_loop(0, NT, drain, 0)
    plsc.subcore_barrier()

    @pl.when(sid < 10)
    def _():
        pltpu.sync_copy(acc.at[pl.ds(sid * 1000, 1000)], zbuf.at[pl.ds(0, 1000)])
        pltpu.sync_copy(zbuf.at[pl.ds(0, 1000)],
                        out.at[pl.ds(cid * N + sid * 1000, 1000)])


def _deg_call(dsti, ew):
    return pl.kernel(
        _deg_body,
        out_type=jax.ShapeDtypeStruct((NC * N,), jnp.float32),
        mesh=_mesh(),
        compiler_params=pltpu.CompilerParams(use_tc_tiling_on_sc=False),
        scratch_types=[
            pltpu.VMEM((NT, CH), jnp.int32),
            pltpu.VMEM((NT, CH), jnp.float32),
            pltpu.VMEM((1024,), jnp.float32),
            pltpu.VMEM_SHARED((N,), jnp.float32),
            pltpu.SemaphoreType.DMA,
        ],
    )(dsti, ew)


NB = 4


def _edge_body(plo, phi, srci, dsti, ew, out0, out1, srcall, dstall, ewall,
               rows, zbuf, stg, acc, g0, g1, g2, g3, s0, s1, s2, s3):
    cid = lax.axis_index("c")
    sid = lax.axis_index("s")
    wid = cid * NS + sid
    phalves = (plo, phi)
    outs = (out0, out1)
    gsems = (g0, g1, g2, g3)
    ssems = (s0, s1, s2, s3)

    pltpu.sync_copy(srci.at[wid], srcall)
    pltpu.sync_copy(dsti.at[wid], dstall)
    pltpu.sync_copy(ew.at[wid], ewall)

    def zrow(r, _):
        for k in range(HH // 16):
            zbuf[r, pl.ds(16 * k, 16)] = jnp.zeros((16,), jnp.float32)
        return 0
    lax.fori_loop(0, ZR, zrow, 0)

    def scale(b, ch):
        @plsc.parallel_loop(0, CH // 16, 1, unroll=4)
        def _sgroup(g):
            wvec = ewall[ch, pl.ds(16 * g, 16)]
            for i in range(16):
                w = wvec[i]
                r = 16 * g + i
                for k in range(HH // 16):
                    sl = pl.ds(16 * k, 16)
                    rows[b, r, sl] = rows[b, r, sl] * w

    for ph in range(2):
        p_hbm = phalves[ph]
        out = outs[ph]

        for j in range(3):
            pltpu.sync_copy(zbuf, acc.at[pl.ds(sid * RPT + j * ZR, ZR)])

        @pl.when(sid == NS - 1)
        def _():
            pltpu.sync_copy(zbuf.at[pl.ds(0, 16)], acc.at[pl.ds(NS * RPT, 16)])
        plsc.subcore_barrier()

        def start_gather(b, ch):
            pltpu.async_copy(p_hbm.at[srcall.at[ch]], rows.at[b], gsems[b])

        def wait_gather(b, ch):
            pltpu.make_async_copy(p_hbm.at[srcall.at[ch]], rows.at[b],
                                  gsems[b]).wait()

        def start_scatter(b, ch):
            pltpu.async_copy(rows.at[b], acc.at[dstall.at[ch]], ssems[b],
                             add=True)

        def wait_scatter(b, ch):
            pltpu.make_async_copy(rows.at[b], acc.at[dstall.at[ch]],
                                  ssems[b]).wait()

        for b in range(NB - 1):
            start_gather(b, b)

        def step(j, _):
            for b in range(NB):
                ch = j * NB + b
                chf = ch + NB - 1
                slotf = (b + NB - 1) % NB

                @pl.when(jnp.logical_and(chf >= NB, chf < NT))
                def _():
                    wait_scatter(slotf, chf - NB)

                @pl.when(chf < NT)
                def _():
                    start_gather(slotf, chf)
                wait_gather(b, ch)
                scale(b, ch)
                start_scatter(b, ch)
            return 0
        lax.fori_loop(0, NT // NB, step, 0)
        for b in range(NB):
            wait_scatter(b, NT - NB + b)
        plsc.subcore_barrier()
        for j in range(3):
            pltpu.sync_copy(acc.at[pl.ds(sid * RPT + j * ZR, ZR)], stg)
            pltpu.sync_copy(stg, out.at[cid, pl.ds(sid * RPT + j * ZR, ZR)])

        @pl.when(sid == NS - 1)
        def _():
            pltpu.sync_copy(acc.at[pl.ds(NS * RPT, 16)], stg.at[pl.ds(0, 16)])
            pltpu.sync_copy(stg.at[pl.ds(0, 16)],
                            out.at[cid, pl.ds(NS * RPT, 16)])


def _edge_call(plo, phi, srci, dsti, ew):
    return pl.kernel(
        _edge_body,
        out_type=[
            jax.ShapeDtypeStruct((NC, N, HH), jnp.float32),
            jax.ShapeDtypeStruct((NC, N, HH), jnp.float32),
        ],
        mesh=_mesh(),
        compiler_params=pltpu.CompilerParams(use_tc_tiling_on_sc=False),
        scratch_types=[
            pltpu.VMEM((NT, CH), jnp.int32),
            pltpu.VMEM((NT, CH), jnp.int32),
            pltpu.VMEM((NT, CH), jnp.float32),
            pltpu.VMEM((NB, CH, HH), jnp.float32),
            pltpu.VMEM((ZR, HH), jnp.float32),
            pltpu.VMEM((ZR, HH), jnp.float32),
            pltpu.VMEM_SHARED((N, HH), jnp.float32),
        ] + [pltpu.SemaphoreType.DMA] * (2 * NB),
    )(plo, phi, srci, dsti, ew)


def _pool_body(h0, h1, h2, batch, maxo, sumo, batchv, hbuf, maxacc, sumacc):
    cid = lax.axis_index("c")
    sid = lax.axis_index("s")
    wid = cid * NS + sid
    base = wid * 320
    hs = (h0, h1, h2)

    neg = jnp.full((16,), -jnp.inf, dtype=jnp.float32)
    zero = jnp.zeros((16,), jnp.float32)

    def irow(r, _):
        for k in range(8):
            sl = pl.ds(16 * k, 16)
            maxacc[r, sl] = neg
            sumacc[r, sl] = zero
        return 0
    lax.fori_loop(0, 3 * G, irow, 0)

    for sub in range(4):
        @pl.when(base + sub * 80 < N)
        def _():
            _pool_sub(batch, hs, batchv, hbuf, maxacc, sumacc, base, sub)

    pltpu.sync_copy(maxacc, maxo.at[wid])
    pltpu.sync_copy(sumacc, sumo.at[wid])


def _pool_sub(batch, hs, batchv, hbuf, maxacc, sumacc, base, sub):
        pltpu.sync_copy(batch.at[pl.ds(base + sub * 80, 80)],
                        batchv.at[pl.ds(sub * 80, 80)])
        for l in range(3):
            pltpu.sync_copy(hs[l].at[pl.ds(base + sub * 80, 80)], hbuf.at[l])

        def rowgrp(g, _):
            bvec = batchv[pl.ds(sub * 80 + 16 * g, 16)]
            for i in range(16):
                b = bvec[i]
                r = 16 * g + i
                for l in range(3):
                    idx = b + l * G
                    for k in range(8):
                        sl = pl.ds(16 * k, 16)
                        hv = hbuf[l, r, sl]
                        maxacc[idx, sl] = jnp.maximum(maxacc[idx, sl], hv)
                        sumacc[idx, sl] = sumacc[idx, sl] + hv
            return 0
        lax.fori_loop(0, 5, rowgrp, 0)


def _pool_call(h0, h1, h2, batch):
    return pl.kernel(
        _pool_body,
        out_type=[
            jax.ShapeDtypeStruct((NW, 3 * G, H), jnp.float32),
            jax.ShapeDtypeStruct((NW, 3 * G, H), jnp.float32),
        ],
        mesh=_mesh(),
        scratch_types=[
            pltpu.VMEM((320,), jnp.int32),
            pltpu.VMEM((3, 80, H), jnp.float32),
            pltpu.VMEM((3 * G, H), jnp.float32),
            pltpu.VMEM((3 * G, H), jnp.float32),
        ],
    )(h0, h1, h2, batch)


def _mlph_body(x_ref, w1_ref, b1_ref, w2_ref, b2_ref, h_ref):
    h = jnp.maximum(x_ref[...] @ w1_ref[...] + b1_ref[...], 0.0)
    h_ref[...] = jnp.maximum(h @ w2_ref[...] + b2_ref[...], 0.0)


def _mlph_call(x, W1, b1, W2, b2):
    return pl.pallas_call(
        _mlph_body,
        grid=(N // RB,),
        in_specs=[
            pl.BlockSpec((RB, F), lambda i: (i, 0)),
            pl.BlockSpec((F, H), lambda i: (0, 0)),
            pl.BlockSpec((1, H), lambda i: (0, 0)),
            pl.BlockSpec((H, H), lambda i: (0, 0)),
            pl.BlockSpec((1, H), lambda i: (0, 0)),
        ],
        out_specs=pl.BlockSpec((RB, H), lambda i: (i, 0)),
        out_shape=jax.ShapeDtypeStruct((N, H), jnp.float32),
    )(x, W1, b1.reshape(1, H), W2, b2.reshape(1, H))


def _proj_body(h_ref, wc_ref, deg_ref, plo_ref, phi_ref, dinv_ref):
    deg = deg_ref[0] + deg_ref[1] + 1.0
    dinv = lax.rsqrt(deg)
    p = dinv * (h_ref[...] @ wc_ref[...])
    plo_ref[...] = p[:, :HH]
    phi_ref[...] = p[:, HH:]
    dinv_ref[...] = dinv


def _proj_call(h, Wc0, deg2):
    return pl.pallas_call(
        _proj_body,
        grid=(N // RB,),
        in_specs=[
            pl.BlockSpec((RB, H), lambda i: (i, 0)),
            pl.BlockSpec((H, H), lambda i: (0, 0)),
            pl.BlockSpec((2, RB, 1), lambda i: (0, i, 0)),
        ],
        out_specs=[
            pl.BlockSpec((RB, HH), lambda i: (i, 0)),
            pl.BlockSpec((RB, HH), lambda i: (i, 0)),
            pl.BlockSpec((RB, 1), lambda i: (i, 0)),
        ],
        out_shape=[
            jax.ShapeDtypeStruct((N, HH), jnp.float32),
            jax.ShapeDtypeStruct((N, HH), jnp.float32),
            jax.ShapeDtypeStruct((N, 1), jnp.float32),
        ],
    )(h, Wc0, deg2)


def _combine_body(plo_part, phi_part, plo_ref, phi_ref, dinv_ref, bc_ref,
                  wcn_ref, h_ref, plon_ref, phin_ref):
    dinv = dinv_ref[...]
    out_lo = dinv * (plo_part[0] + plo_part[1] + plo_ref[...]) + bc_ref[:, :HH]
    out_hi = dinv * (phi_part[0] + phi_part[1] + phi_ref[...]) + bc_ref[:, HH:]
    h = jnp.concatenate([jnp.maximum(out_lo, 0.0), jnp.maximum(out_hi, 0.0)],
                        axis=1)
    h_ref[...] = h
    pn = dinv * (h @ wcn_ref[...])
    plon_ref[...] = pn[:, :HH]
    phin_ref[...] = pn[:, HH:]


def _combine_call(part_lo, part_hi, plo, phi, dinv, bc, wcn):
    return pl.pallas_call(
        _combine_body,
        grid=(N // RB,),
        in_specs=[
            pl.BlockSpec((2, RB, HH), lambda i: (0, i, 0)),
            pl.BlockSpec((2, RB, HH), lambda i: (0, i, 0)),
            pl.BlockSpec((RB, HH), lambda i: (i, 0)),
            pl.BlockSpec((RB, HH), lambda i: (i, 0)),
            pl.BlockSpec((RB, 1), lambda i: (i, 0)),
            pl.BlockSpec((1, H), lambda i: (0, 0)),
            pl.BlockSpec((H, H), lambda i: (0, 0)),
        ],
        out_specs=[
            pl.BlockSpec((RB, H), lambda i: (i, 0)),
            pl.BlockSpec((RB, HH), lambda i: (i, 0)),
            pl.BlockSpec((RB, HH), lambda i: (i, 0)),
        ],
        out_shape=[
            jax.ShapeDtypeStruct((N, H), jnp.float32),
            jax.ShapeDtypeStruct((N, HH), jnp.float32),
            jax.ShapeDtypeStruct((N, HH), jnp.float32),
        ],
    )(part_lo, part_hi, plo, phi, dinv, bc.reshape(1, H), wcn)


def _poolcomb_body(maxp_ref, sump_ref, batch_ref, out_ref):
    mx = jnp.max(maxp_ref[...], axis=0)
    mx = jnp.where(jnp.isfinite(mx), mx, 0.0)
    sums = jnp.sum(sump_ref[...], axis=0)
    ids = lax.broadcasted_iota(jnp.int32, (1, G), 1)
    cnt = jnp.sum((batch_ref[...] == ids).astype(jnp.float32), axis=0)
    mean = sums / jnp.maximum(cnt, 1.0)[:, None]
    out_ref[...] = jnp.concatenate([mx, mean], axis=1)


def _poolcomb_call(maxp, sump, batch2):
    return pl.pallas_call(
        _poolcomb_body,
        grid=(3,),
        in_specs=[
            pl.BlockSpec((NW, G, H), lambda l: (0, l, 0)),
            pl.BlockSpec((NW, G, H), lambda l: (0, l, 0)),
            pl.BlockSpec((N, 1), lambda l: (0, 0)),
        ],
        out_specs=pl.BlockSpec((G, 2 * H), lambda l: (0, l)),
        out_shape=jax.ShapeDtypeStruct((G, 6 * H), jnp.float32),
    )(maxp, sump, batch2)


def kernel(x, edge_index, edge_attr, batch, W1, b1, W2, b2, Wc0, bc0,
           Wc1, bc1, Wc2, bc2):
    pad = EPAD - E
    pidx = jnp.arange(pad, dtype=jnp.int32) % N
    srci = jnp.concatenate([edge_index[0], pidx])
    dsti = jnp.concatenate([edge_index[1], pidx])
    ewp = jnp.concatenate([edge_attr, jnp.zeros((pad,), jnp.float32)])
    srci3 = srci.reshape(NW, NT, CH)
    dsti3 = dsti.reshape(NW, NT, CH)
    ewp3 = ewp.reshape(NW, NT, CH)

    h0 = _mlph_call(x, W1, b1, W2, b2)
    deg2 = _deg_call(dsti3, ewp3).reshape(NC, N, 1)
    plo, phi, dinv = _proj_call(h0, Wc0, deg2)

    hs = []
    for bc, wcn in ((bc0, Wc1), (bc1, Wc2), (bc2, Wc2)):
        part_lo, part_hi = _edge_call(plo, phi, srci3, dsti3, ewp3)
        h, plo, phi = _combine_call(part_lo, part_hi, plo, phi, dinv, bc, wcn)
        hs.append(h)

    maxp, sump = _pool_call(hs[0], hs[1], hs[2], batch)
    return _poolcomb_call(maxp, sump, batch.reshape(N, 1))
